# baseline (speedup 1.0000x reference)
import functools

import jax
import jax.numpy as jnp
from jax import lax
from jax.experimental import pallas as pl
from jax.experimental.pallas import tpu as pltpu
from jax.experimental.pallas import tpu_sc as plsc

NUM_MODELS = 2
NUM_BOXES = 100000
DIMS = 64
BATCH = 16384
ROW = 2 * DIMS

NC = 2
NS = 16
NW = NC * NS
B_PER_W = BATCH // NW
CHUNK = 128
N_CHUNKS = B_PER_W // CHUNK
LANES = 16


@functools.partial(
    pl.kernel,
    mesh=plsc.VectorSubcoreMesh(core_axis_name="c", subcore_axis_name="s"),
    out_type=jax.ShapeDtypeStruct((NUM_MODELS * BATCH, ROW), jnp.float32),
    scratch_types=[
        pltpu.VMEM((N_CHUNKS, CHUNK), jnp.int32),
        pltpu.VMEM((N_CHUNKS, CHUNK), jnp.int32),
        pltpu.VMEM((2, CHUNK, ROW), jnp.float32),
        pltpu.SemaphoreType.DMA,
        pltpu.SemaphoreType.DMA,
    ],
)
def _gather_rows(table_hbm, idx_hbm, out_hbm, idx_v, idx2_v, rows_v, gsem0, gsem1):
    wid = lax.axis_index("s") * NC + lax.axis_index("c")
    base = wid * B_PER_W

    pltpu.sync_copy(idx_hbm.at[pl.ds(wid * N_CHUNKS, N_CHUNKS)], idx_v)

    for j in range(N_CHUNKS):
        for k in range(CHUNK // LANES):
            sl = pl.ds(k * LANES, LANES)
            idx2_v[j, sl] = idx_v[j, sl] + NUM_BOXES

    sems = (gsem0, gsem1)

    def chunk_idx_ref(t):
        m, j = divmod(t, N_CHUNKS)
        return idx_v.at[j] if m == 0 else idx2_v.at[j]

    def out_base(t):
        m, j = divmod(t, N_CHUNKS)
        return m * BATCH + base + j * CHUNK

    total = NUM_MODELS * N_CHUNKS
    pending = pltpu.async_copy(table_hbm.at[chunk_idx_ref(0)], rows_v.at[0], sems[0])
    for t in range(total):
        buf = t % 2
        nbuf = (t + 1) % 2
        pending.wait()
        if t + 1 < total:
            nxt = pltpu.async_copy(
                table_hbm.at[chunk_idx_ref(t + 1)], rows_v.at[nbuf], sems[nbuf])
        pltpu.sync_copy(rows_v.at[buf], out_hbm.at[pl.ds(out_base(t), CHUNK)])
        if t + 1 < total:
            pending = nxt


def kernel(boxes, box_indices):
    table = boxes.reshape(NUM_MODELS * NUM_BOXES, ROW)
    idx = box_indices.astype(jnp.int32).reshape(BATCH // CHUNK, CHUNK)
    out = _gather_rows(table, idx)
    return out.reshape(NUM_MODELS, BATCH, 2, DIMS)


# speedup vs baseline: 4.5480x; 4.5480x over previous
import functools

import jax
import jax.numpy as jnp
from jax import lax
from jax.experimental import pallas as pl
from jax.experimental.pallas import tpu as pltpu
from jax.experimental.pallas import tpu_sc as plsc

NUM_MODELS = 2
NUM_BOXES = 100000
DIMS = 64
BATCH = 16384

R = NUM_MODELS * 2 * DIMS
NC = 2
NS = 16
NW = NC * NS
ROWS_PER_W = R // NW
LANES = 16
OCHUNK = 4096
N_OCHUNKS = BATCH // OCHUNK


@functools.partial(
    pl.kernel,
    mesh=plsc.VectorSubcoreMesh(core_axis_name="c", subcore_axis_name="s"),
    compiler_params=pltpu.CompilerParams(needs_layout_passes=False),
    out_type=jax.ShapeDtypeStruct((R, BATCH), jnp.float32),
    scratch_types=[
        pltpu.VMEM((NUM_BOXES,), jnp.float32),
        pltpu.VMEM((BATCH // 128, 128), jnp.int32),
        pltpu.VMEM((2, OCHUNK), jnp.float32),
        pltpu.SemaphoreType.DMA,
        pltpu.SemaphoreType.DMA,
    ],
)
def _plane_gather(table_hbm, idx_hbm, out_hbm, row_v, idx_v, out_v, rsem, osem):
    wid = lax.axis_index("s") * NC + lax.axis_index("c")

    pltpu.sync_copy(idx_hbm, idx_v)

    for k in range(ROWS_PER_W):
        r = wid * ROWS_PER_W + k
        pltpu.sync_copy(table_hbm.at[r], row_v)

        for h in range(N_OCHUNKS):
            buf = h % 2

            def gather_chunk(c, _, h=h, buf=buf):
                i = h * (OCHUNK // LANES) + c
                idx16 = idx_v[i >> 3, pl.ds((i & 7) * LANES, LANES)]
                out_v[buf, pl.ds(c * LANES, LANES)] = plsc.load_gather(
                    row_v, [idx16])
                return _

            lax.fori_loop(0, OCHUNK // LANES, gather_chunk, 0)
            if h > 0:
                prev.wait()
            prev = pltpu.async_copy(
                out_v.at[buf], out_hbm.at[r, pl.ds(h * OCHUNK, OCHUNK)], osem)
        prev.wait()


def kernel(boxes, box_indices):
    table = boxes.transpose(0, 2, 3, 1).reshape(R, NUM_BOXES)
    idx = box_indices.astype(jnp.int32).reshape(BATCH // 128, 128)
    out = _plane_gather(table, idx)
    return out.reshape(NUM_MODELS, 2, DIMS, BATCH).transpose(0, 3, 1, 2)
